# Initial kernel scaffold; baseline (speedup 1.0000x reference)
import jax
import jax.numpy as jnp
from jax.experimental import pallas as pl


def kernel(x, edge_index, Wih0, Whh0, bih0, bhh0, Wih1, Whh1, bih1, bhh1, gW1, gb1, gW2, gb2, lw0, lb0, lw1, lb1, lw2, lb2, lw3, lb3):
    raise NotImplementedError("write your pallas kernel here")


# speedup vs baseline: 1707.5143x; 1707.5143x over previous
import jax
import jax.numpy as jnp
from jax.experimental import pallas as pl

_B, _N, _T, _F = 8, 500, 32, 16
_H = 16
_E = 10000
_BN = _B * _N
_ECH = 2000


def _body(xT_ref, dst_ref, srcT_ref, Wc0_ref, bih0_ref, bhh0_ref,
          Wc1_ref, bih1_ref, bhh1_ref,
          gW1_ref, gb1_ref, gW2_ref, gb2_ref,
          L0_ref, lb0_ref, L1_ref, lb1_ref, L2_ref, lb2_ref, L3_ref, lb3_ref,
          out_ref):
    f32 = jnp.float32

    def mm(a, b):
        return jax.lax.dot_general(a, b, (((1,), (0,)), ((), ())),
                                   preferred_element_type=f32)

    m23 = mm(L2_ref[...], L3_ref[...])
    m123 = mm(L1_ref[...], m23)
    m = mm(L0_ref[...], m123)
    g2m = mm(gW2_ref[...], m)
    w_fold = mm(gW1_ref[...], g2m)
    alpha = mm(gb1_ref[...], g2m)
    c_mlp = (mm(lb0_ref[...], m123) + mm(lb1_ref[...], m23)
             + mm(lb2_ref[...], L3_ref[...]) + lb3_ref[...])
    beta = mm(gb2_ref[...], m) + c_mlp

    adj = jnp.zeros((_N, _N), f32)
    for k in range(_E // _ECH):
        dst = dst_ref[0:1, k * _ECH:(k + 1) * _ECH]
        src = srcT_ref[k * _ECH:(k + 1) * _ECH, 0:1]
        iota_d = jax.lax.broadcasted_iota(jnp.int32, (_N, _ECH), 0)
        oh_dstT = (iota_d == dst).astype(jnp.bfloat16)
        iota_s = jax.lax.broadcasted_iota(jnp.int32, (_ECH, _N), 1)
        oh_src = (iota_s == src).astype(jnp.bfloat16)
        adj = adj + mm(oh_dstT, oh_src)

    eye = (jax.lax.broadcasted_iota(jnp.int32, (_N, _N), 0)
           == jax.lax.broadcasted_iota(jnp.int32, (_N, _N), 1)).astype(f32)
    deg_col = jnp.sum(adj, axis=1, keepdims=True) + 2.0
    deg_row = jnp.sum(eye * deg_col, axis=0, keepdims=True)
    dinv_row = jax.lax.rsqrt(deg_row)
    dinv_col = jax.lax.rsqrt(deg_col)
    ahat = adj * dinv_col * dinv_row + eye * (2.0 / deg_col)
    r_col = jnp.sum(ahat, axis=1, keepdims=True)
    a2 = mm(ahat, ahat)

    Wc0 = Wc0_ref[...]
    Wc1 = Wc1_ref[...]
    b0 = bih0_ref[...] + bhh0_ref[...]
    b1 = bih1_ref[...] + bhh1_ref[...]
    wT = jnp.transpose(w_fold)

    h1 = jnp.zeros((_H, _BN), f32)
    c1 = jnp.zeros((_H, _BN), f32)
    h2 = jnp.zeros((_H, _BN), f32)
    c2 = jnp.zeros((_H, _BN), f32)
    ys = []
    for t in range(_T):
        xt = xT_ref[t]
        S = jnp.concatenate([xt, h1], axis=0)
        Gt = mm(Wc0, S) + b0
        ig = jax.nn.sigmoid(Gt[0:_H])
        fg = jax.nn.sigmoid(Gt[_H:2 * _H])
        gg = jnp.tanh(Gt[2 * _H:3 * _H])
        og = jax.nn.sigmoid(Gt[3 * _H:4 * _H])
        c1 = fg * c1 + ig * gg
        h1 = og * jnp.tanh(c1)

        S2 = jnp.concatenate([h1, h2], axis=0)
        G2 = mm(Wc1, S2) + b1
        i2 = jax.nn.sigmoid(G2[0:_H])
        f2 = jax.nn.sigmoid(G2[_H:2 * _H])
        g2 = jnp.tanh(G2[2 * _H:3 * _H])
        o2 = jax.nn.sigmoid(G2[3 * _H:4 * _H])
        c2 = f2 * c2 + i2 * g2
        h2 = o2 * jnp.tanh(c2)

        ys.append(mm(wT, h2))

    Y = jnp.concatenate(ys, axis=0)
    Yr = jnp.transpose(Y)
    Yb = jnp.reshape(Yr, (_B, _N, _T))

    zs = [mm(a2, Yb[b])[None] for b in range(_B)]
    out = jnp.concatenate(zs, axis=0) + alpha[0, 0] * r_col + beta[0, 0]
    out_ref[...] = out


def kernel(x, edge_index, Wih0, Whh0, bih0, bhh0, Wih1, Whh1, bih1, bhh1,
           gW1, gb1, gW2, gb2, lw0, lb0, lw1, lb1, lw2, lb2, lw3, lb3):
    xT = jnp.transpose(x, (2, 3, 0, 1)).reshape(_T, _F, _BN)
    ei = edge_index.astype(jnp.int32)
    dst = ei[1].reshape(1, _E)
    srcT = ei[0].reshape(_E, 1)
    Wc0 = jnp.concatenate([Wih0, Whh0], axis=1)
    Wc1 = jnp.concatenate([Wih1, Whh1], axis=1)

    out = pl.pallas_call(
        _body,
        out_shape=jax.ShapeDtypeStruct((_B, _N, _T), jnp.float32),
    )(xT, dst, srcT, Wc0,
      bih0.reshape(4 * _H, 1), bhh0.reshape(4 * _H, 1),
      Wc1, bih1.reshape(4 * _H, 1), bhh1.reshape(4 * _H, 1),
      gW1, gb1.reshape(1, _H), gW2, gb2.reshape(1, 16),
      lw0.T, lb0.reshape(1, 16), lw1.T, lb1.reshape(1, 8),
      lw2.T, lb2.reshape(1, 4), lw3.T, lb3.reshape(1, 1))

    return out


# speedup vs baseline: 1831.7580x; 1.0728x over previous
import jax
import jax.numpy as jnp
from jax.experimental import pallas as pl

_B, _N, _T, _F = 8, 500, 32, 16
_H = 16
_E = 10000
_BN = _B * _N
_ECH = 2000


def _body(xT_ref, dst_ref, srcT_ref, Wc0_ref, bih0_ref, bhh0_ref,
          Wc1_ref, bih1_ref, bhh1_ref,
          gW1_ref, gb1_ref, gW2_ref, gb2_ref,
          L0_ref, lb0_ref, L1_ref, lb1_ref, L2_ref, lb2_ref, L3_ref, lb3_ref,
          out_ref):
    f32 = jnp.float32

    def mm(a, b):
        return jax.lax.dot_general(a, b, (((1,), (0,)), ((), ())),
                                   preferred_element_type=f32)

    m23 = mm(L2_ref[...], L3_ref[...])
    m123 = mm(L1_ref[...], m23)
    m = mm(L0_ref[...], m123)
    g2m = mm(gW2_ref[...], m)
    w_fold = mm(gW1_ref[...], g2m)
    alpha = mm(gb1_ref[...], g2m)
    c_mlp = (mm(lb0_ref[...], m123) + mm(lb1_ref[...], m23)
             + mm(lb2_ref[...], L3_ref[...]) + lb3_ref[...])
    beta = mm(gb2_ref[...], m) + c_mlp

    adj = jnp.zeros((_N, _N), f32)
    for k in range(_E // _ECH):
        dst = dst_ref[0:1, k * _ECH:(k + 1) * _ECH]
        src = srcT_ref[k * _ECH:(k + 1) * _ECH, 0:1]
        iota_d = jax.lax.broadcasted_iota(jnp.int32, (_N, _ECH), 0)
        oh_dstT = (iota_d == dst).astype(jnp.bfloat16)
        iota_s = jax.lax.broadcasted_iota(jnp.int32, (_ECH, _N), 1)
        oh_src = (iota_s == src).astype(jnp.bfloat16)
        adj = adj + mm(oh_dstT, oh_src)

    eye = (jax.lax.broadcasted_iota(jnp.int32, (_N, _N), 0)
           == jax.lax.broadcasted_iota(jnp.int32, (_N, _N), 1)).astype(f32)
    deg_col = jnp.sum(adj, axis=1, keepdims=True) + 2.0
    deg_row = jnp.sum(eye * deg_col, axis=0, keepdims=True)
    dinv_row = jax.lax.rsqrt(deg_row)
    dinv_col = jax.lax.rsqrt(deg_col)
    ahat = adj * dinv_col * dinv_row + eye * (2.0 / deg_col)
    r_col = jnp.sum(ahat, axis=1, keepdims=True)
    a2 = mm(ahat, ahat)

    Wc0 = Wc0_ref[...]
    Wc1 = Wc1_ref[...]
    b0 = bih0_ref[...] + bhh0_ref[...]
    b1 = bih1_ref[...] + bhh1_ref[...]
    wT = jnp.transpose(w_fold)

    def sig(v):
        return 0.5 * jnp.tanh(0.5 * v) + 0.5

    h1 = jnp.zeros((_H, _BN), f32)
    c1 = jnp.zeros((_H, _BN), f32)
    h2 = jnp.zeros((_H, _BN), f32)
    c2 = jnp.zeros((_H, _BN), f32)
    ys = []
    for t in range(_T):
        xt = xT_ref[t]
        S = jnp.concatenate([xt, h1], axis=0)
        Gt = mm(Wc0, S) + b0
        ig = sig(Gt[0:_H])
        fg = sig(Gt[_H:2 * _H])
        gg = jnp.tanh(Gt[2 * _H:3 * _H])
        og = sig(Gt[3 * _H:4 * _H])
        c1 = fg * c1 + ig * gg
        h1 = og * jnp.tanh(c1)

        S2 = jnp.concatenate([h1, h2], axis=0)
        G2 = mm(Wc1, S2) + b1
        i2 = sig(G2[0:_H])
        f2 = sig(G2[_H:2 * _H])
        g2 = jnp.tanh(G2[2 * _H:3 * _H])
        o2 = sig(G2[3 * _H:4 * _H])
        c2 = f2 * c2 + i2 * g2
        h2 = o2 * jnp.tanh(c2)

        ys.append(mm(wT, h2))

    Y = jnp.concatenate(ys, axis=0)
    Yr = jnp.transpose(Y)
    Yb = jnp.reshape(Yr, (_B, _N, _T))

    zs = [mm(a2, Yb[b])[None] for b in range(_B)]
    out = jnp.concatenate(zs, axis=0) + alpha[0, 0] * r_col + beta[0, 0]
    out_ref[...] = out


def kernel(x, edge_index, Wih0, Whh0, bih0, bhh0, Wih1, Whh1, bih1, bhh1,
           gW1, gb1, gW2, gb2, lw0, lb0, lw1, lb1, lw2, lb2, lw3, lb3):
    xT = jnp.transpose(x, (2, 3, 0, 1)).reshape(_T, _F, _BN)
    ei = edge_index.astype(jnp.int32)
    dst = ei[1].reshape(1, _E)
    srcT = ei[0].reshape(_E, 1)
    Wc0 = jnp.concatenate([Wih0, Whh0], axis=1)
    Wc1 = jnp.concatenate([Wih1, Whh1], axis=1)

    out = pl.pallas_call(
        _body,
        out_shape=jax.ShapeDtypeStruct((_B, _N, _T), jnp.float32),
    )(xT, dst, srcT, Wc0,
      bih0.reshape(4 * _H, 1), bhh0.reshape(4 * _H, 1),
      Wc1, bih1.reshape(4 * _H, 1), bhh1.reshape(4 * _H, 1),
      gW1, gb1.reshape(1, _H), gW2, gb2.reshape(1, 16),
      lw0.T, lb0.reshape(1, 16), lw1.T, lb1.reshape(1, 8),
      lw2.T, lb2.reshape(1, 4), lw3.T, lb3.reshape(1, 1))

    return out
